# Initial kernel scaffold; baseline (speedup 1.0000x reference)
import jax
import jax.numpy as jnp
from jax.experimental import pallas as pl


def kernel(h, edge_index, alphas):
    raise NotImplementedError("write your pallas kernel here")


# speedup vs baseline: 8.8511x; 8.8511x over previous
import functools

import jax
import jax.numpy as jnp
from jax import lax
from jax.experimental import pallas as pl
from jax.experimental.pallas import tpu as pltpu
from jax.experimental.pallas import tpu_sc as plsc

N = 10000
D = 128
E = 320000
NC, NS = 2, 16
NW = NC * NS
EPW = E // NW
CHUNK = 80
NCHUNK = EPW // CHUNK
NPAD = 10240
RPT = NPAD // NS
WB = 128
DEG_PAD = 10240
DPT = DEG_PAD // NS

_mesh = plsc.VectorSubcoreMesh(core_axis_name="c", subcore_axis_name="s")


@functools.partial(
    pl.kernel,
    out_type=jax.ShapeDtypeStruct((NC, DEG_PAD), jnp.float32),
    mesh=_mesh,
    scratch_types=[
        pltpu.VMEM_SHARED((DEG_PAD,), jnp.float32),
        pltpu.VMEM((CHUNK,), jnp.int32),
        pltpu.VMEM((CHUNK,), jnp.float32),
        pltpu.VMEM((128,), jnp.float32),
    ],
)
def _deg_kernel(dst_hbm, out_hbm, acc, didx, ones, stage):
    cid = lax.axis_index("c")
    sid = lax.axis_index("s")
    wid = sid * NC + cid

    for i in range(8):
        stage[pl.ds(i * 16, 16)] = jnp.zeros((16,), jnp.float32)
    for i in range(CHUNK // 16):
        ones[pl.ds(i * 16, 16)] = jnp.ones((16,), jnp.float32)
    for j in range(DPT // 128):
        pltpu.sync_copy(stage, acc.at[pl.ds(sid * DPT + j * 128, 128)])
    plsc.subcore_barrier()

    def body(c, carry):
        base = wid * EPW + c * CHUNK
        pltpu.sync_copy(dst_hbm.at[pl.ds(base, CHUNK)], didx)
        pltpu.sync_copy(ones, acc.at[didx], add=True)
        return carry

    lax.fori_loop(0, NCHUNK, body, 0)
    plsc.subcore_barrier()

    for j in range(DPT // 128):
        pltpu.sync_copy(acc.at[pl.ds(sid * DPT + j * 128, 128)], stage)
        pltpu.sync_copy(stage, out_hbm.at[cid, pl.ds(sid * DPT + j * 128, 128)])


@functools.partial(
    pl.kernel,
    out_type=jax.ShapeDtypeStruct((NC, NPAD, D), jnp.float32),
    mesh=_mesh,
    scratch_types=[
        pltpu.VMEM_SHARED((NPAD, D), jnp.float32),
        pltpu.VMEM((CHUNK,), jnp.int32),
        pltpu.VMEM((CHUNK,), jnp.int32),
        pltpu.VMEM((CHUNK, D), jnp.float32),
        pltpu.VMEM((WB, D), jnp.float32),
        pltpu.SemaphoreType.DMA,
    ],
)
def _prop_kernel(z_hbm, src_hbm, dst_hbm, out_hbm, acc, sidx, didx, rows, wb, sem):
    cid = lax.axis_index("c")
    sid = lax.axis_index("s")
    wid = sid * NC + cid

    def zrow(i, carry):
        wb[i // 8, pl.ds((i % 8) * 16, 16)] = jnp.zeros((16,), jnp.float32)
        return carry

    lax.fori_loop(0, WB * 8, zrow, 0)
    for j in range(RPT // WB):
        pltpu.sync_copy(wb, acc.at[pl.ds(sid * RPT + j * WB, WB)])
    plsc.subcore_barrier()

    def body(c, carry):
        base = wid * EPW + c * CHUNK
        pltpu.sync_copy(src_hbm.at[pl.ds(base, CHUNK)], sidx)
        pltpu.sync_copy(dst_hbm.at[pl.ds(base, CHUNK)], didx)
        pltpu.async_copy(z_hbm.at[sidx], rows, sem).wait()
        pltpu.sync_copy(rows, acc.at[didx], add=True)
        return carry

    lax.fori_loop(0, NCHUNK, body, 0)
    plsc.subcore_barrier()

    for j in range(RPT // WB):
        pltpu.sync_copy(acc.at[pl.ds(sid * RPT + j * WB, WB)], wb)
        pltpu.sync_copy(wb, out_hbm.at[cid, pl.ds(sid * RPT + j * WB, WB)])


def kernel(h, edge_index, alphas):
    src = edge_index[0]
    dst = edge_index[1]

    deg_p = _deg_kernel(dst)
    deg = deg_p[0, :N] + deg_p[1, :N]
    dvec = jnp.where(deg > 0, lax.rsqrt(jnp.maximum(deg, 1.0)), 0.0)
    d1 = dvec[:, None]
    d2 = d1 * d1

    w1 = _prop_kernel(d1 * h, src, dst)
    w1 = w1[0, :N] + w1[1, :N]
    w2 = _prop_kernel(d2 * w1, src, dst)
    w2 = w2[0, :N] + w2[1, :N]
    w3 = _prop_kernel(d2 * w2, src, dst)
    w3 = w3[0, :N] + w3[1, :N]

    powers = [h, d1 * w1, d1 * w2, d1 * w3]

    outs = [h]
    idx = 0
    for i in range(3):
        coef = [jnp.float32(1.0)]
        for _ in range(i + 1):
            a = alphas[idx]
            idx += 1
            new = [(1 - a) * coef[0]]
            for k in range(1, len(coef)):
                new.append((1 - a) * coef[k] + a * coef[k - 1])
            new.append(a * coef[-1])
            coef = new
        outs.append(sum(c * p for c, p in zip(coef, powers)))
    return jnp.concatenate(outs, axis=1)


# speedup vs baseline: 18.8745x; 2.1325x over previous
import functools

import jax
import jax.numpy as jnp
from jax import lax
from jax.experimental import pallas as pl
from jax.experimental.pallas import tpu as pltpu
from jax.experimental.pallas import tpu_sc as plsc

N = 10000
D = 128
E = 320000
NC, NS = 2, 16
NW = NC * NS
EPW = E // NW
CHUNK = 80
NCHUNK = EPW // CHUNK
NPAD = 10240
RPT = NPAD // NS
DEG_PAD = 10240
DPT = DEG_PAD // NS

_mesh = plsc.VectorSubcoreMesh(core_axis_name="c", subcore_axis_name="s")


@functools.partial(
    pl.kernel,
    out_type=jax.ShapeDtypeStruct((NC, DEG_PAD), jnp.float32),
    mesh=_mesh,
    scratch_types=[
        pltpu.VMEM_SHARED((DEG_PAD,), jnp.float32),
        pltpu.VMEM((CHUNK,), jnp.int32),
        pltpu.VMEM((CHUNK,), jnp.float32),
        pltpu.VMEM((128,), jnp.float32),
    ],
)
def _deg_kernel(dst_hbm, out_hbm, acc, didx, ones, stage):
    cid = lax.axis_index("c")
    sid = lax.axis_index("s")
    wid = sid * NC + cid

    for i in range(8):
        stage[pl.ds(i * 16, 16)] = jnp.zeros((16,), jnp.float32)
    for i in range(CHUNK // 16):
        ones[pl.ds(i * 16, 16)] = jnp.ones((16,), jnp.float32)
    for j in range(DPT // 128):
        pltpu.sync_copy(stage, acc.at[pl.ds(sid * DPT + j * 128, 128)])
    plsc.subcore_barrier()

    def body(c, carry):
        base = wid * EPW + c * CHUNK
        pltpu.sync_copy(dst_hbm.at[pl.ds(base, CHUNK)], didx)
        pltpu.sync_copy(ones, acc.at[didx], add=True)
        return carry

    lax.fori_loop(0, NCHUNK, body, 0)
    plsc.subcore_barrier()

    for j in range(DPT // 128):
        pltpu.sync_copy(acc.at[pl.ds(sid * DPT + j * 128, 128)], stage)
        pltpu.sync_copy(stage, out_hbm.at[cid, pl.ds(sid * DPT + j * 128, 128)])


@functools.partial(
    pl.kernel,
    out_type=jax.ShapeDtypeStruct((NC, NPAD, D), jnp.float32),
    mesh=_mesh,
    scratch_types=[
        pltpu.VMEM_SHARED((NPAD, D), jnp.float32),
        pltpu.VMEM((EPW,), jnp.int32),
        pltpu.VMEM((EPW,), jnp.int32),
        pltpu.VMEM((CHUNK, D), jnp.float32),
        pltpu.VMEM((CHUNK, D), jnp.float32),
        pltpu.SemaphoreType.DMA,
        pltpu.SemaphoreType.DMA,
    ],
)
def _prop_kernel(z_hbm, src_hbm, dst_hbm, out_hbm, acc, sidx, didx,
                 rows0, rows1, sem0, sem1):
    cid = lax.axis_index("c")
    sid = lax.axis_index("s")
    wid = sid * NC + cid

    pltpu.sync_copy(src_hbm.at[pl.ds(wid * EPW, EPW)], sidx)
    pltpu.sync_copy(dst_hbm.at[pl.ds(wid * EPW, EPW)], didx)

    def zrow(i, carry):
        rows0[i // 8, pl.ds((i % 8) * 16, 16)] = jnp.zeros((16,), jnp.float32)
        return carry

    lax.fori_loop(0, CHUNK * 8, zrow, 0)
    for j in range(RPT // CHUNK):
        pltpu.sync_copy(rows0, acc.at[pl.ds(sid * RPT + j * CHUNK, CHUNK)])
    plsc.subcore_barrier()

    bufs = ((rows0, sem0), (rows1, sem1))

    pltpu.async_copy(z_hbm.at[sidx.at[pl.ds(0, CHUNK)]], rows0, sem0)
    pltpu.async_copy(z_hbm.at[sidx.at[pl.ds(CHUNK, CHUNK)]], rows1, sem1)

    @pl.loop(0, NCHUNK - 1, step=2)
    def _pipeline(c0):
        for b in range(2):
            c = c0 + b
            rows, sem = bufs[b]
            pltpu.make_async_copy(
                z_hbm.at[sidx.at[pl.ds(c * CHUNK, CHUNK)]], rows, sem).wait()
            pltpu.sync_copy(rows, acc.at[didx.at[pl.ds(c * CHUNK, CHUNK)]],
                            add=True)

            @pl.when(c + 2 < NCHUNK)
            def _():
                pltpu.async_copy(
                    z_hbm.at[sidx.at[pl.ds((c + 2) * CHUNK, CHUNK)]], rows, sem)

    last = NCHUNK - 1
    rows, sem = bufs[last % 2]
    pltpu.make_async_copy(
        z_hbm.at[sidx.at[pl.ds(last * CHUNK, CHUNK)]], rows, sem).wait()
    pltpu.sync_copy(rows, acc.at[didx.at[pl.ds(last * CHUNK, CHUNK)]], add=True)

    plsc.subcore_barrier()

    for j in range(RPT // CHUNK):
        pltpu.sync_copy(acc.at[pl.ds(sid * RPT + j * CHUNK, CHUNK)], rows0)
        pltpu.sync_copy(rows0, out_hbm.at[cid, pl.ds(sid * RPT + j * CHUNK, CHUNK)])


def kernel(h, edge_index, alphas):
    src = edge_index[0]
    dst = edge_index[1]

    deg_p = _deg_kernel(dst)
    deg = deg_p[0, :N] + deg_p[1, :N]
    dvec = jnp.where(deg > 0, lax.rsqrt(jnp.maximum(deg, 1.0)), 0.0)
    d1 = dvec[:, None]
    d2 = d1 * d1

    w1 = _prop_kernel(d1 * h, src, dst)
    w1 = w1[0, :N] + w1[1, :N]
    w2 = _prop_kernel(d2 * w1, src, dst)
    w2 = w2[0, :N] + w2[1, :N]
    w3 = _prop_kernel(d2 * w2, src, dst)
    w3 = w3[0, :N] + w3[1, :N]

    powers = [h, d1 * w1, d1 * w2, d1 * w3]

    outs = [h]
    idx = 0
    for i in range(3):
        coef = [jnp.float32(1.0)]
        for _ in range(i + 1):
            a = alphas[idx]
            idx += 1
            new = [(1 - a) * coef[0]]
            for k in range(1, len(coef)):
                new.append((1 - a) * coef[k] + a * coef[k - 1])
            new.append(a * coef[-1])
            coef = new
        outs.append(sum(c * p for c, p in zip(coef, powers)))
    return jnp.concatenate(outs, axis=1)


# speedup vs baseline: 22.7638x; 1.2061x over previous
import functools

import jax
import jax.numpy as jnp
from jax import lax
from jax.experimental import pallas as pl
from jax.experimental.pallas import tpu as pltpu
from jax.experimental.pallas import tpu_sc as plsc

N = 10000
D = 128
E = 320000
NC, NS = 2, 16
NW = NC * NS
EPW0 = E // NW
PADE = 8
EPW = EPW0 + PADE
CHUNK = 72
NCHUNK = EPW // CHUNK
NBUF = 3
NPAD = 10240
RPT = NPAD // NS
DEG_PAD = 10240
DPT = DEG_PAD // NS

_mesh = plsc.VectorSubcoreMesh(core_axis_name="c", subcore_axis_name="s")


@functools.partial(
    pl.kernel,
    out_type=jax.ShapeDtypeStruct((NC, DEG_PAD), jnp.float32),
    mesh=_mesh,
    scratch_types=[
        pltpu.VMEM_SHARED((DEG_PAD,), jnp.float32),
        pltpu.VMEM((EPW,), jnp.int32),
        pltpu.VMEM((80,), jnp.float32),
        pltpu.VMEM((128,), jnp.float32),
        pltpu.SemaphoreType.DMA,
    ],
)
def _deg_kernel(dst_hbm, out_hbm, acc, didx, ones, stage, sem):
    cid = lax.axis_index("c")
    sid = lax.axis_index("s")
    wid = sid * NC + cid

    pltpu.sync_copy(dst_hbm.at[pl.ds(wid * EPW, EPW)], didx)
    for i in range(8):
        stage[pl.ds(i * 16, 16)] = jnp.zeros((16,), jnp.float32)
    for i in range(5):
        ones[pl.ds(i * 16, 16)] = jnp.ones((16,), jnp.float32)
    for j in range(DPT // 128):
        pltpu.sync_copy(stage, acc.at[pl.ds(sid * DPT + j * 128, 128)])
    plsc.subcore_barrier()

    @pl.loop(0, NCHUNK)
    def _fire(c):
        pltpu.async_copy(ones.at[pl.ds(0, CHUNK)],
                         acc.at[didx.at[pl.ds(c * CHUNK, CHUNK)]],
                         sem, add=True)

        @pl.when(c >= 4)
        def _():
            pltpu.make_async_copy(
                ones.at[pl.ds(0, CHUNK)],
                acc.at[didx.at[pl.ds(0, CHUNK)]], sem).wait()

    @pl.loop(0, 4)
    def _drain(c):
        pltpu.make_async_copy(
            ones.at[pl.ds(0, CHUNK)],
            acc.at[didx.at[pl.ds(0, CHUNK)]], sem).wait()

    plsc.subcore_barrier()

    for j in range(DPT // 128):
        pltpu.sync_copy(acc.at[pl.ds(sid * DPT + j * 128, 128)], stage)
        pltpu.sync_copy(stage, out_hbm.at[cid, pl.ds(sid * DPT + j * 128, 128)])


@functools.partial(
    pl.kernel,
    out_type=jax.ShapeDtypeStruct((NC, NPAD, D), jnp.float32),
    mesh=_mesh,
    scratch_types=[
        pltpu.VMEM_SHARED((NPAD, D), jnp.float32),
        pltpu.VMEM((EPW,), jnp.int32),
        pltpu.VMEM((EPW,), jnp.int32),
        pltpu.VMEM((CHUNK, D), jnp.float32),
        pltpu.VMEM((CHUNK, D), jnp.float32),
        pltpu.VMEM((CHUNK, D), jnp.float32),
        pltpu.SemaphoreType.DMA,
        pltpu.SemaphoreType.DMA,
        pltpu.SemaphoreType.DMA,
        pltpu.SemaphoreType.DMA,
        pltpu.SemaphoreType.DMA,
        pltpu.SemaphoreType.DMA,
    ],
)
def _prop_kernel(z_hbm, src_hbm, dst_hbm, out_hbm, acc, sidx, didx,
                 rows0, rows1, rows2, gs0, gs1, gs2, ss0, ss1, ss2):
    cid = lax.axis_index("c")
    sid = lax.axis_index("s")
    wid = sid * NC + cid

    rows = (rows0, rows1, rows2)
    gsem = (gs0, gs1, gs2)
    ssem = (ss0, ss1, ss2)

    pltpu.sync_copy(src_hbm.at[pl.ds(wid * EPW, EPW)], sidx)
    pltpu.sync_copy(dst_hbm.at[pl.ds(wid * EPW, EPW)], didx)

    def zrow(i, carry):
        rows0[i // 8, pl.ds((i % 8) * 16, 16)] = jnp.zeros((16,), jnp.float32)
        return carry

    lax.fori_loop(0, CHUNK * 8, zrow, 0)
    nfull = RPT // CHUNK
    tail = RPT - nfull * CHUNK
    for j in range(nfull):
        pltpu.sync_copy(rows0, acc.at[pl.ds(sid * RPT + j * CHUNK, CHUNK)])
    pltpu.sync_copy(rows0.at[pl.ds(0, tail)],
                    acc.at[pl.ds(sid * RPT + nfull * CHUNK, tail)])
    plsc.subcore_barrier()

    def gather(c, b):
        pltpu.async_copy(
            z_hbm.at[sidx.at[pl.ds(c * CHUNK, CHUNK)]], rows[b], gsem[b])

    def scatter(c, b):
        pltpu.async_copy(
            rows[b], acc.at[didx.at[pl.ds(c * CHUNK, CHUNK)]], ssem[b],
            add=True)

    def wait_gather(c, b):
        pltpu.make_async_copy(
            z_hbm.at[sidx.at[pl.ds(c * CHUNK, CHUNK)]], rows[b],
            gsem[b]).wait()

    def wait_scatter(b):
        pltpu.make_async_copy(
            rows[b], acc.at[didx.at[pl.ds(0, CHUNK)]], ssem[b]).wait()

    gather(0, 0)
    gather(1, 1)

    @pl.loop(0, NCHUNK - 1, step=NBUF)
    def _pipeline(c0):
        for b in range(NBUF):
            c = c0 + b
            wait_gather(c, b)
            scatter(c, b)

            @pl.when(c + 2 < NCHUNK)
            def _():
                nb = (b + 2) % NBUF

                @pl.when(c > 0)
                def _():
                    wait_scatter(nb)

                gather(c + 2, nb)

    last = NCHUNK - 1
    wait_gather(last, last % NBUF)
    scatter(last, last % NBUF)
    for b in range(NBUF):
        wait_scatter(b)

    plsc.subcore_barrier()

    for j in range(nfull):
        pltpu.sync_copy(acc.at[pl.ds(sid * RPT + j * CHUNK, CHUNK)], rows0)
        pltpu.sync_copy(rows0, out_hbm.at[cid, pl.ds(sid * RPT + j * CHUNK,
                                                     CHUNK)])
    pltpu.sync_copy(acc.at[pl.ds(sid * RPT + nfull * CHUNK, tail)],
                    rows0.at[pl.ds(0, tail)])
    pltpu.sync_copy(rows0.at[pl.ds(0, tail)],
                    out_hbm.at[cid, pl.ds(sid * RPT + nfull * CHUNK, tail)])


def kernel(h, edge_index, alphas):
    src = edge_index[0]
    dst = edge_index[1]

    srcp = jnp.concatenate(
        [src.reshape(NW, EPW0), jnp.zeros((NW, PADE), jnp.int32)],
        axis=1).reshape(-1)
    dstp = jnp.concatenate(
        [dst.reshape(NW, EPW0), jnp.full((NW, PADE), N, jnp.int32)],
        axis=1).reshape(-1)

    deg_p = _deg_kernel(dstp)
    deg = deg_p[0, :N] + deg_p[1, :N]
    dvec = jnp.where(deg > 0, lax.rsqrt(jnp.maximum(deg, 1.0)), 0.0)
    d1 = dvec[:, None]
    d2 = d1 * d1

    w1 = _prop_kernel(d1 * h, srcp, dstp)
    w1 = w1[0, :N] + w1[1, :N]
    w2 = _prop_kernel(d2 * w1, srcp, dstp)
    w2 = w2[0, :N] + w2[1, :N]
    w3 = _prop_kernel(d2 * w2, srcp, dstp)
    w3 = w3[0, :N] + w3[1, :N]

    powers = [h, d1 * w1, d1 * w2, d1 * w3]

    outs = [h]
    idx = 0
    for i in range(3):
        coef = [jnp.float32(1.0)]
        for _ in range(i + 1):
            a = alphas[idx]
            idx += 1
            new = [(1 - a) * coef[0]]
            for k in range(1, len(coef)):
                new.append((1 - a) * coef[k] + a * coef[k - 1])
            new.append(a * coef[-1])
            coef = new
        outs.append(sum(c * p for c, p in zip(coef, powers)))
    return jnp.concatenate(outs, axis=1)


# speedup vs baseline: 23.3045x; 1.0238x over previous
import functools

import jax
import jax.numpy as jnp
from jax import lax
from jax.experimental import pallas as pl
from jax.experimental.pallas import tpu as pltpu
from jax.experimental.pallas import tpu_sc as plsc

N = 10000
D = 128
E = 320000
NC, NS = 2, 16
NW = NC * NS
EPW0 = E // NW
PADE = 8
EPW = EPW0 + PADE
CHUNK = 72
NCHUNK = EPW // CHUNK
NBUF = 3
NPAD = 10240
RPT = NPAD // NS
DEG_PAD = 10240
DPT = DEG_PAD // NS

_mesh = plsc.VectorSubcoreMesh(core_axis_name="c", subcore_axis_name="s")


@functools.partial(
    pl.kernel,
    out_type=jax.ShapeDtypeStruct((NC, DEG_PAD), jnp.float32),
    mesh=_mesh,
    scratch_types=[
        pltpu.VMEM_SHARED((DEG_PAD,), jnp.float32),
        pltpu.VMEM((EPW,), jnp.int32),
        pltpu.VMEM((80,), jnp.float32),
        pltpu.VMEM((128,), jnp.float32),
        pltpu.SemaphoreType.DMA,
    ],
)
def _deg_kernel(dst_hbm, out_hbm, acc, didx, ones, stage, sem):
    cid = lax.axis_index("c")
    sid = lax.axis_index("s")
    wid = sid * NC + cid

    pltpu.sync_copy(dst_hbm.at[pl.ds(wid * EPW, EPW)], didx)
    for i in range(8):
        stage[pl.ds(i * 16, 16)] = jnp.zeros((16,), jnp.float32)
    for i in range(5):
        ones[pl.ds(i * 16, 16)] = jnp.ones((16,), jnp.float32)
    for j in range(DPT // 128):
        pltpu.sync_copy(stage, acc.at[pl.ds(sid * DPT + j * 128, 128)])
    plsc.subcore_barrier()

    @pl.loop(0, NCHUNK)
    def _fire(c):
        pltpu.async_copy(ones.at[pl.ds(0, CHUNK)],
                         acc.at[didx.at[pl.ds(c * CHUNK, CHUNK)]],
                         sem, add=True)

        @pl.when(c >= 4)
        def _():
            pltpu.make_async_copy(
                ones.at[pl.ds(0, CHUNK)],
                acc.at[didx.at[pl.ds(0, CHUNK)]], sem).wait()

    @pl.loop(0, 4)
    def _drain(c):
        pltpu.make_async_copy(
            ones.at[pl.ds(0, CHUNK)],
            acc.at[didx.at[pl.ds(0, CHUNK)]], sem).wait()

    plsc.subcore_barrier()

    for j in range(DPT // 128):
        pltpu.sync_copy(acc.at[pl.ds(sid * DPT + j * 128, 128)], stage)
        pltpu.sync_copy(stage, out_hbm.at[cid, pl.ds(sid * DPT + j * 128, 128)])


@functools.partial(
    pl.kernel,
    out_type=jax.ShapeDtypeStruct((NC, NPAD, D), jnp.float32),
    mesh=_mesh,
    scratch_types=[
        pltpu.VMEM_SHARED((NPAD, D), jnp.float32),
        pltpu.VMEM((EPW,), jnp.int32),
        pltpu.VMEM((EPW,), jnp.int32),
        pltpu.VMEM((CHUNK, D), jnp.float32),
        pltpu.VMEM((CHUNK, D), jnp.float32),
        pltpu.VMEM((CHUNK, D), jnp.float32),
        pltpu.SemaphoreType.DMA,
        pltpu.SemaphoreType.DMA,
        pltpu.SemaphoreType.DMA,
        pltpu.SemaphoreType.DMA,
        pltpu.SemaphoreType.DMA,
        pltpu.SemaphoreType.DMA,
    ],
)
def _prop_kernel(z_hbm, src_hbm, dst_hbm, out_hbm, acc, sidx, didx,
                 rows0, rows1, rows2, gs0, gs1, gs2, ss0, ss1, ss2):
    cid = lax.axis_index("c")
    sid = lax.axis_index("s")
    wid = sid * NC + cid

    rows = (rows0, rows1, rows2)
    gsem = (gs0, gs1, gs2)
    ssem = (ss0, ss1, ss2)

    def zrow(i, carry):
        rows2[i // 8, pl.ds((i % 8) * 16, 16)] = jnp.zeros((16,), jnp.float32)
        return carry

    lax.fori_loop(0, CHUNK * 8, zrow, 0)
    nfull = RPT // CHUNK
    tail = RPT - nfull * CHUNK
    for j in range(nfull):
        pltpu.async_copy(rows2, acc.at[pl.ds(sid * RPT + j * CHUNK, CHUNK)],
                         ss0)
    pltpu.async_copy(rows2.at[pl.ds(0, tail)],
                     acc.at[pl.ds(sid * RPT + nfull * CHUNK, tail)], ss0)

    pltpu.sync_copy(src_hbm.at[pl.ds(wid * EPW, EPW)], sidx)
    pltpu.sync_copy(dst_hbm.at[pl.ds(wid * EPW, EPW)], didx)

    for j in range(nfull):
        pltpu.make_async_copy(
            rows2, acc.at[pl.ds(sid * RPT + j * CHUNK, CHUNK)], ss0).wait()
    pltpu.make_async_copy(
        rows2.at[pl.ds(0, tail)],
        acc.at[pl.ds(sid * RPT + nfull * CHUNK, tail)], ss0).wait()
    plsc.subcore_barrier()

    def gather(c, b):
        pltpu.async_copy(
            z_hbm.at[sidx.at[pl.ds(c * CHUNK, CHUNK)]], rows[b], gsem[b])

    def scatter(c, b):
        pltpu.async_copy(
            rows[b], acc.at[didx.at[pl.ds(c * CHUNK, CHUNK)]], ssem[b],
            add=True)

    def wait_gather(c, b):
        pltpu.make_async_copy(
            z_hbm.at[sidx.at[pl.ds(c * CHUNK, CHUNK)]], rows[b],
            gsem[b]).wait()

    def wait_scatter(b):
        pltpu.make_async_copy(
            rows[b], acc.at[didx.at[pl.ds(0, CHUNK)]], ssem[b]).wait()

    gather(0, 0)
    gather(1, 1)

    @pl.loop(0, NCHUNK - 1, step=NBUF)
    def _pipeline(c0):
        for b in range(NBUF):
            c = c0 + b
            wait_gather(c, b)
            scatter(c, b)

            @pl.when(c + 2 < NCHUNK)
            def _():
                nb = (b + 2) % NBUF

                @pl.when(c > 0)
                def _():
                    wait_scatter(nb)

                gather(c + 2, nb)

    last = NCHUNK - 1
    wait_gather(last, last % NBUF)
    scatter(last, last % NBUF)
    for b in range(NBUF):
        wait_scatter(b)

    plsc.subcore_barrier()

    pltpu.sync_copy(acc.at[pl.ds(sid * RPT, RPT)],
                    out_hbm.at[cid, pl.ds(sid * RPT, RPT)])


def kernel(h, edge_index, alphas):
    src = edge_index[0]
    dst = edge_index[1]

    srcp = jnp.concatenate(
        [src.reshape(NW, EPW0), jnp.zeros((NW, PADE), jnp.int32)],
        axis=1).reshape(-1)
    dstp = jnp.concatenate(
        [dst.reshape(NW, EPW0), jnp.full((NW, PADE), N, jnp.int32)],
        axis=1).reshape(-1)

    deg_p = _deg_kernel(dstp)
    deg = deg_p[0, :N] + deg_p[1, :N]
    dvec = jnp.where(deg > 0, lax.rsqrt(jnp.maximum(deg, 1.0)), 0.0)
    d1 = dvec[:, None]
    d2 = d1 * d1

    w1 = _prop_kernel(d1 * h, srcp, dstp)
    w1 = w1[0, :N] + w1[1, :N]
    w2 = _prop_kernel(d2 * w1, srcp, dstp)
    w2 = w2[0, :N] + w2[1, :N]
    w3 = _prop_kernel(d2 * w2, srcp, dstp)
    w3 = w3[0, :N] + w3[1, :N]

    powers = [h, d1 * w1, d1 * w2, d1 * w3]

    outs = [h]
    idx = 0
    for i in range(3):
        coef = [jnp.float32(1.0)]
        for _ in range(i + 1):
            a = alphas[idx]
            idx += 1
            new = [(1 - a) * coef[0]]
            for k in range(1, len(coef)):
                new.append((1 - a) * coef[k] + a * coef[k - 1])
            new.append(a * coef[-1])
            coef = new
        outs.append(sum(c * p for c, p in zip(coef, powers)))
    return jnp.concatenate(outs, axis=1)
